# baseline (speedup 1.0000x reference)
import functools

import jax
import jax.numpy as jnp
from jax import lax
from jax.experimental import pallas as pl
from jax.experimental.pallas import tpu as pltpu
from jax.experimental.pallas import tpu_sc as plsc

B = 16384
D = 64
NC = 2
NS = 16
NW = NC * NS
CHUNK = B // NW
GSZ = 128
NG = CHUNK // GSZ
BLK = 16
NBLK = CHUNK // BLK


def _score_block(h_rows, r_rows, t_rows, out_v, b):
    iota16 = lax.iota(jnp.int32, 16)
    res = None
    for rr in range(BLK):
        row = b * BLK + rr
        acc = None
        for s in range(D // 16):
            sl = pl.ds(s * 16, 16)
            hv = h_rows[row, sl]
            rv = r_rows[row, sl]
            tv = t_rows[row, sl]
            dv = (hv - tv) + rv
            sq = dv * dv
            acc = sq if acc is None else acc + sq
        for k in (8, 4, 2, 1):
            acc = acc + jnp.take(acc, iota16 ^ k)
        res = (jnp.where(iota16 == rr, acc, jnp.float32(0.0)) if res is None
               else jnp.where(iota16 == rr, acc, res))
    x = res + jnp.float32(1e-12)
    m = x
    s = jnp.float32(1.0)
    for k in (16, 8, 4, 2, 1):
        c = m < jnp.float32(4.0 ** (-k))
        m = jnp.where(c, m * jnp.float32(4.0 ** k), m)
        s = jnp.where(c, s * jnp.float32(2.0 ** k), s)
    y = s * (jnp.float32(7.0 / 3.0) - jnp.float32(4.0 / 3.0) * m)
    for _ in range(4):
        y = y * (jnp.float32(1.5) - jnp.float32(0.5) * x * y * y)
    out_v[pl.ds(b * BLK, BLK)] = -(x * y)


def _body(ent_hbm, rel_hbm, h_idx_hbm, r_idx_hbm, t_idx_hbm, out_hbm,
          h_idx_v, r_idx_v, t_idx_v, h_rows, r_rows, t_rows, out_v, sem):
    wid = lax.axis_index("s") * NC + lax.axis_index("c")
    pltpu.sync_copy(h_idx_hbm.at[wid], h_idx_v)
    pltpu.sync_copy(r_idx_hbm.at[wid], r_idx_v)
    pltpu.sync_copy(t_idx_hbm.at[wid], t_idx_v)
    copies = []
    for j in range(NG):
        dst = pl.ds(j * GSZ, GSZ)
        copies.append(pltpu.async_copy(ent_hbm.at[h_idx_v.at[j]], h_rows.at[dst], sem))
        copies.append(pltpu.async_copy(ent_hbm.at[t_idx_v.at[j]], t_rows.at[dst], sem))
        copies.append(pltpu.async_copy(rel_hbm.at[r_idx_v.at[j]], r_rows.at[dst], sem))
    for c in copies:
        c.wait()

    def block_body(b, carry):
        _score_block(h_rows, r_rows, t_rows, out_v, b)
        return carry

    lax.fori_loop(0, NBLK, block_body, 0)
    pltpu.sync_copy(out_v, out_hbm.at[wid])


_sc_call = functools.partial(
    pl.kernel,
    out_type=jax.ShapeDtypeStruct((NW, CHUNK), jnp.float32),
    mesh=plsc.VectorSubcoreMesh(core_axis_name="c", subcore_axis_name="s"),
    compiler_params=pltpu.CompilerParams(use_tc_tiling_on_sc=False),
    scratch_types=[
        pltpu.VMEM((NG, GSZ), jnp.int32),
        pltpu.VMEM((NG, GSZ), jnp.int32),
        pltpu.VMEM((NG, GSZ), jnp.int32),
        pltpu.VMEM((CHUNK, D), jnp.float32),
        pltpu.VMEM((CHUNK, D), jnp.float32),
        pltpu.VMEM((CHUNK, D), jnp.float32),
        pltpu.VMEM((CHUNK,), jnp.float32),
        pltpu.SemaphoreType.DMA,
    ],
)(_body)


def kernel(ent_emb, rel_emb, batch_h, batch_r, batch_t):
    h3 = batch_h.reshape(NW, NG, GSZ)
    r3 = batch_r.reshape(NW, NG, GSZ)
    t3 = batch_t.reshape(NW, NG, GSZ)
    out = _sc_call(ent_emb, rel_emb, h3, r3, t3)
    return out.reshape(B)


# speedup vs baseline: 1.6655x; 1.6655x over previous
import functools

import jax
import jax.numpy as jnp
from jax import lax
from jax.experimental import pallas as pl
from jax.experimental.pallas import tpu as pltpu
from jax.experimental.pallas import tpu_sc as plsc

B = 16384
D = 64
NC = 2
NS = 16
NW = NC * NS
CHUNK = B // NW
BLK = 16
CROWS = 128
NCH = CHUNK // CROWS
CBLK = CROWS // BLK


def _score_block(h_rows, r_rows, t_rows, out_v, slot, c, b):
    iota16 = lax.iota(jnp.int32, 16)
    res = None
    for rr in range(BLK):
        row = b * BLK + rr
        acc = None
        for s in range(D // 16):
            sl = pl.ds(s * 16, 16)
            hv = h_rows[slot, row, sl]
            rv = r_rows[slot, row, sl]
            tv = t_rows[slot, row, sl]
            dv = (hv - tv) + rv
            sq = dv * dv
            acc = sq if acc is None else acc + sq
        for k in (8, 4, 2, 1):
            acc = acc + jnp.take(acc, iota16 ^ k)
        res = (jnp.where(iota16 == rr, acc, jnp.float32(0.0)) if res is None
               else jnp.where(iota16 == rr, acc, res))
    x = res + jnp.float32(1e-12)
    m = x
    s = jnp.float32(1.0)
    for k in (16, 8, 4, 2, 1):
        cond = m < jnp.float32(4.0 ** (-k))
        m = jnp.where(cond, m * jnp.float32(4.0 ** k), m)
        s = jnp.where(cond, s * jnp.float32(2.0 ** k), s)
    y = s * (jnp.float32(7.0 / 3.0) - jnp.float32(4.0 / 3.0) * m)
    for _ in range(4):
        y = y * (jnp.float32(1.5) - jnp.float32(0.5) * x * y * y)
    out_v[pl.ds(c * CROWS + b * BLK, BLK)] = -(x * y)


def _body(ent_hbm, rel_hbm, h_idx_hbm, r_idx_hbm, t_idx_hbm, out_hbm,
          h_idx_v, r_idx_v, t_idx_v, h_rows, r_rows, t_rows, out_v,
          sem0, sem1):
    wid = lax.axis_index("s") * NC + lax.axis_index("c")
    pltpu.sync_copy(h_idx_hbm.at[wid], h_idx_v)
    pltpu.sync_copy(r_idx_hbm.at[wid], r_idx_v)
    pltpu.sync_copy(t_idx_hbm.at[wid], t_idx_v)
    sems = (sem0, sem1)

    def enqueue_chunk(c, slot):
        def blk(b, carry):
            off = c * CROWS + b * BLK
            hv = h_idx_v[pl.ds(off, BLK)]
            rv = r_idx_v[pl.ds(off, BLK)]
            tv = t_idx_v[pl.ds(off, BLK)]
            for l in range(BLK):
                row = b * BLK + l
                pltpu.async_copy(ent_hbm.at[hv[l]], h_rows.at[slot, row], sems[slot])
                pltpu.async_copy(ent_hbm.at[tv[l]], t_rows.at[slot, row], sems[slot])
                pltpu.async_copy(rel_hbm.at[rv[l]], r_rows.at[slot, row], sems[slot])
            return carry

        lax.fori_loop(0, CBLK, blk, 0)

    def drain_chunk(slot):
        dummy = ent_hbm.at[pl.ds(0, CROWS)]
        pltpu.make_async_copy(dummy, h_rows.at[slot], sems[slot]).wait()
        pltpu.make_async_copy(dummy, t_rows.at[slot], sems[slot]).wait()
        pltpu.make_async_copy(dummy, r_rows.at[slot], sems[slot]).wait()

    enqueue_chunk(0, 0)
    for c in range(NCH):
        slot = c % 2
        if c + 1 < NCH:
            enqueue_chunk(c + 1, 1 - slot)
        drain_chunk(slot)

        def block_body(b, carry, slot=slot, c=c):
            _score_block(h_rows, r_rows, t_rows, out_v, slot, c, b)
            return carry

        lax.fori_loop(0, CBLK, block_body, 0)
    pltpu.sync_copy(out_v, out_hbm.at[wid])


_sc_call = functools.partial(
    pl.kernel,
    out_type=jax.ShapeDtypeStruct((NW, CHUNK), jnp.float32),
    mesh=plsc.VectorSubcoreMesh(core_axis_name="c", subcore_axis_name="s"),
    compiler_params=pltpu.CompilerParams(use_tc_tiling_on_sc=True),
    scratch_types=[
        pltpu.VMEM((CHUNK,), jnp.int32),
        pltpu.VMEM((CHUNK,), jnp.int32),
        pltpu.VMEM((CHUNK,), jnp.int32),
        pltpu.VMEM((2, CROWS, D), jnp.float32),
        pltpu.VMEM((2, CROWS, D), jnp.float32),
        pltpu.VMEM((2, CROWS, D), jnp.float32),
        pltpu.VMEM((CHUNK,), jnp.float32),
        pltpu.SemaphoreType.DMA,
        pltpu.SemaphoreType.DMA,
    ],
)(_body)


def kernel(ent_emb, rel_emb, batch_h, batch_r, batch_t):
    h2 = batch_h.reshape(NW, CHUNK)
    r2 = batch_r.reshape(NW, CHUNK)
    t2 = batch_t.reshape(NW, CHUNK)
    out = _sc_call(ent_emb, rel_emb, h2, r2, t2)
    return out.reshape(B)
